# baseline (speedup 1.0000x reference)
import functools

import jax
import jax.numpy as jnp
from jax import lax
from jax.experimental import pallas as pl
from jax.experimental.pallas import tpu as pltpu
from jax.experimental.pallas import tpu_sc as plsc

N = 10000
NP = 10240
E = 320000
C = 128
H = 256

_INFO = plsc.get_sparse_core_info()
NC = _INFO.num_cores
NS = _INFO.num_subcores
NW = NC * NS
K = 128
RPW = 80
ROWS = NW * RPW
EP = ROWS * K
BLK = 4
NBLK = RPW // BLK
RPT = NP // NS

_MESH = plsc.VectorSubcoreMesh(core_axis_name="c", subcore_axis_name="s")


def _worker_id():
    cid = lax.axis_index("c")
    sid = lax.axis_index("s")
    return cid, sid, sid * NC + cid


@functools.partial(
    pl.kernel,
    out_type=jax.ShapeDtypeStruct((NC * NP,), jnp.float32),
    mesh=_MESH,
    scratch_types=[
        pltpu.VMEM((RPW, K), jnp.int32),
        pltpu.VMEM((K,), jnp.float32),
        pltpu.VMEM((RPT,), jnp.float32),
        pltpu.VMEM_SHARED((NP,), jnp.float32),
        pltpu.SemaphoreType.DMA,
    ],
)
def _deg_kernel(dst_hbm, out_hbm, didx, onesv, zb, acc, dsem):
    cid, sid, wid = _worker_id()
    zvec = jnp.zeros((16,), jnp.float32)
    ovec = jnp.ones((16,), jnp.float32)

    def fill(i, _):
        zb[pl.ds(i * 16, 16)] = zvec
        return 0

    lax.fori_loop(0, RPT // 16, fill, 0)

    def fill1(i, _):
        onesv[pl.ds(i * 16, 16)] = ovec
        return 0

    lax.fori_loop(0, K // 16, fill1, 0)
    pltpu.sync_copy(dst_hbm.at[pl.ds(wid * RPW, RPW)], didx)
    pltpu.sync_copy(zb, acc.at[pl.ds(sid * RPT, RPT)])
    plsc.subcore_barrier()

    @pl.loop(0, RPW, step=8)
    def chunk(c):
        for j in range(8):
            pltpu.async_copy(onesv, acc.at[didx.at[c + j]], add=True,
                             sem=dsem)
        for j in range(8):
            pltpu.make_async_copy(onesv, acc.at[didx.at[c + j]], dsem).wait()

    plsc.subcore_barrier()
    pltpu.sync_copy(acc.at[pl.ds(sid * RPT, RPT)], zb)
    pltpu.sync_copy(zb, out_hbm.at[pl.ds(cid * NP + sid * RPT, RPT)])


@functools.partial(
    pl.kernel,
    out_type=jax.ShapeDtypeStruct((NC, NP, C), jnp.float32),
    mesh=_MESH,
    scratch_types=[
        pltpu.VMEM((2 * BLK, K), jnp.int32),
        pltpu.VMEM((2 * BLK, K), jnp.int32),
        pltpu.VMEM((K, C), jnp.float32),
        pltpu.VMEM((K, C), jnp.float32),
        pltpu.VMEM_SHARED((NP, C), jnp.float32),
        pltpu.SemaphoreType.DMA,
        pltpu.SemaphoreType.DMA,
        pltpu.SemaphoreType.DMA,
        pltpu.SemaphoreType.DMA,
        pltpu.SemaphoreType.DMA,
        pltpu.SemaphoreType.DMA,
    ],
)
def _agg_kernel(u_hbm, ei_hbm, out_hbm, idx0, idx1, rows0, rows1,
                acc, isem0, isem1, gsem0, gsem1, ssem0, ssem1):
    cid, sid, wid = _worker_id()
    zvec = jnp.zeros((16,), jnp.float32)
    rbufs = (rows0, rows1)
    gsems = (gsem0, gsem1)
    ssems = (ssem0, ssem1)

    def fill(i, _):
        for q in range(8):
            rows0[i, pl.ds(q * 16, 16)] = zvec
        return 0

    lax.fori_loop(0, K, fill, 0)

    for j in range(RPT // K):
        pltpu.async_copy(rows0, acc.at[pl.ds(sid * RPT + j * K, K)], gsem1)
    for j in range(RPT // K):
        pltpu.make_async_copy(rows0, acc.at[pl.ds(sid * RPT + j * K, K)],
                              gsem1).wait()
    plsc.subcore_barrier()

    base2 = wid * RPW * 2

    def idx_slice(b):
        return ei_hbm.at[pl.ds(base2 + b * 2 * BLK, 2 * BLK)]

    def one_block(b, ix, iy, isy):
        for j in range(BLK):
            cur = rbufs[j % 2]
            nxt = rbufs[(j + 1) % 2]
            pltpu.make_async_copy(u_hbm.at[ix.at[2 * j]], cur,
                                  gsems[j % 2]).wait()
            pltpu.async_copy(cur, acc.at[ix.at[2 * j + 1]], ssems[j % 2],
                             add=True)
            if j == 0:
                @pl.when(b >= 1)
                def _():
                    pltpu.make_async_copy(nxt, acc.at[iy.at[2 * BLK - 1]],
                                          ssems[1]).wait()

                @pl.when(b + 1 < NBLK)
                def _():
                    pltpu.async_copy(idx_slice(b + 1), iy, isy)
            else:
                pltpu.make_async_copy(nxt, acc.at[ix.at[2 * (j - 1) + 1]],
                                      ssems[(j + 1) % 2]).wait()
            if j < BLK - 1:
                pltpu.async_copy(u_hbm.at[ix.at[2 * (j + 1)]], nxt,
                                 gsems[(j + 1) % 2])
            else:
                @pl.when(b + 1 < NBLK)
                def _():
                    pltpu.make_async_copy(idx_slice(b + 1), iy, isy).wait()
                    pltpu.async_copy(u_hbm.at[iy.at[0]], nxt,
                                     gsems[(j + 1) % 2])

    pltpu.sync_copy(idx_slice(0), idx0)
    pltpu.async_copy(u_hbm.at[idx0.at[0]], rows0, gsem0)

    @pl.loop(0, NBLK, step=2)
    def blocks(b):
        one_block(b, idx0, idx1, isem1)
        one_block(b + 1, idx1, idx0, isem0)

    pltpu.make_async_copy(rbufs[1], acc.at[idx1.at[2 * BLK - 1]],
                          ssems[1]).wait()
    plsc.subcore_barrier()

    row0 = sid * RPT
    pltpu.async_copy(acc.at[pl.ds(row0, K)], rows0, gsem0)
    for j in range(RPT // K):
        cur = rbufs[j % 2]
        if j + 1 < RPT // K:
            pltpu.async_copy(acc.at[pl.ds(row0 + (j + 1) * K, K)],
                             rbufs[(j + 1) % 2], gsems[(j + 1) % 2])
        pltpu.make_async_copy(acc.at[pl.ds(row0 + j * K, K)], cur,
                              gsems[j % 2]).wait()
        pltpu.sync_copy(cur, out_hbm.at[cid, pl.ds(row0 + j * K, K)])


_BS = 1024


def _scale_body(degp_ref, x_ref, d_ref, u1_ref):
    deg = degp_ref[0] + degp_ref[1] + 1.0
    d = lax.rsqrt(deg)
    d_ref[...] = d
    u1_ref[...] = x_ref[...] * d


def _scale_call(degp, x):
    return pl.pallas_call(
        _scale_body,
        grid=(NP // _BS,),
        in_specs=[
            pl.BlockSpec((NC, _BS, 1), lambda i: (0, i, 0)),
            pl.BlockSpec((_BS, C), lambda i: (i, 0)),
        ],
        out_specs=[
            pl.BlockSpec((_BS, 1), lambda i: (i, 0)),
            pl.BlockSpec((_BS, C), lambda i: (i, 0)),
        ],
        out_shape=[
            jax.ShapeDtypeStruct((NP, 1), jnp.float32),
            jax.ShapeDtypeStruct((NP, C), jnp.float32),
        ],
    )(degp, x)


def _dense_body(s1p_ref, u1_ref, d_ref, W1_ref, b1_ref, W2_ref, u2_ref):
    t = s1p_ref[0] + s1p_ref[1] + u1_ref[...]
    d = d_ref[...]
    y = jnp.dot(t.astype(jnp.bfloat16), W1_ref[...].astype(jnp.bfloat16),
                preferred_element_type=jnp.float32) * d
    r = jnp.maximum(y + b1_ref[...], 0.0)
    u2_ref[...] = jnp.dot(r.astype(jnp.bfloat16),
                          W2_ref[...].astype(jnp.bfloat16),
                          preferred_element_type=jnp.float32) * d


def _dense_call(s1p, u1, d, W1, b1, W2):
    return pl.pallas_call(
        _dense_body,
        grid=(NP // _BS,),
        in_specs=[
            pl.BlockSpec((NC, _BS, C), lambda i: (0, i, 0)),
            pl.BlockSpec((_BS, C), lambda i: (i, 0)),
            pl.BlockSpec((_BS, 1), lambda i: (i, 0)),
            pl.BlockSpec((C, H), lambda i: (0, 0)),
            pl.BlockSpec((1, H), lambda i: (0, 0)),
            pl.BlockSpec((H, C), lambda i: (0, 0)),
        ],
        out_specs=pl.BlockSpec((_BS, C), lambda i: (i, 0)),
        out_shape=jax.ShapeDtypeStruct((NP, C), jnp.float32),
    )(s1p, u1, d, W1, b1.reshape(1, H), W2)


def _final_body(s2p_ref, u2_ref, d_ref, b2_ref, out_ref):
    out_ref[...] = ((s2p_ref[0] + s2p_ref[1] + u2_ref[...]) * d_ref[...]
                    + b2_ref[...])


def _final_call(s2p, u2, d, b2):
    return pl.pallas_call(
        _final_body,
        grid=(NP // _BS,),
        in_specs=[
            pl.BlockSpec((NC, _BS, C), lambda i: (0, i, 0)),
            pl.BlockSpec((_BS, C), lambda i: (i, 0)),
            pl.BlockSpec((_BS, 1), lambda i: (i, 0)),
            pl.BlockSpec((1, C), lambda i: (0, 0)),
        ],
        out_specs=pl.BlockSpec((_BS, C), lambda i: (i, 0)),
        out_shape=jax.ShapeDtypeStruct((NP, C), jnp.float32),
    )(s2p, u2, d, b2.reshape(1, C))


def kernel(x, edge_index, W1, b1, W2, b2):
    pad = N + jnp.arange(EP - E, dtype=jnp.int32) % (NP - N)
    src = jnp.concatenate([edge_index[0].astype(jnp.int32), pad]).reshape(ROWS, K)
    dst = jnp.concatenate([edge_index[1].astype(jnp.int32), pad]).reshape(ROWS, K)
    ei = jnp.stack([src, dst], axis=1).reshape(2 * ROWS, K)
    xp = jnp.pad(x, ((0, NP - N), (0, 0)))
    degp = _deg_kernel(dst).reshape(NC, NP, 1)
    d, u1 = _scale_call(degp, xp)
    s1p = _agg_kernel(u1, ei)
    u2 = _dense_call(s1p, u1, d, W1, b1, W2)
    s2p = _agg_kernel(u2, ei)
    return _final_call(s2p, u2, d, b2)[:N]


# speedup vs baseline: 1.1167x; 1.1167x over previous
import functools

import jax
import jax.numpy as jnp
from jax import lax
from jax.experimental import pallas as pl
from jax.experimental.pallas import tpu as pltpu
from jax.experimental.pallas import tpu_sc as plsc

N = 10000
NP = 10240
E = 320000
C = 128
H = 256

_INFO = plsc.get_sparse_core_info()
NC = _INFO.num_cores
NS = _INFO.num_subcores
NW = NC * NS
K = 128
RPW = 80
ROWS = NW * RPW
EP = ROWS * K
BLK = 8
NBLK = RPW // BLK
RPT = NP // NS

_MESH = plsc.VectorSubcoreMesh(core_axis_name="c", subcore_axis_name="s")


def _worker_id():
    cid = lax.axis_index("c")
    sid = lax.axis_index("s")
    return cid, sid, sid * NC + cid


@functools.partial(
    pl.kernel,
    out_type=jax.ShapeDtypeStruct((NC * NP,), jnp.float32),
    mesh=_MESH,
    scratch_types=[
        pltpu.VMEM((RPW, K), jnp.int32),
        pltpu.VMEM((K,), jnp.float32),
        pltpu.VMEM((RPT,), jnp.float32),
        pltpu.VMEM_SHARED((NP,), jnp.float32),
        pltpu.SemaphoreType.DMA,
    ],
)
def _deg_kernel(dst_hbm, out_hbm, didx, onesv, zb, acc, dsem):
    cid, sid, wid = _worker_id()
    zvec = jnp.zeros((16,), jnp.float32)
    ovec = jnp.ones((16,), jnp.float32)

    def fill(i, _):
        zb[pl.ds(i * 16, 16)] = zvec
        return 0

    lax.fori_loop(0, RPT // 16, fill, 0)

    def fill1(i, _):
        onesv[pl.ds(i * 16, 16)] = ovec
        return 0

    lax.fori_loop(0, K // 16, fill1, 0)
    pltpu.sync_copy(dst_hbm.at[pl.ds(wid * RPW, RPW)], didx)
    pltpu.sync_copy(zb, acc.at[pl.ds(sid * RPT, RPT)])
    plsc.subcore_barrier()

    @pl.loop(0, RPW, step=8)
    def chunk(c):
        for j in range(8):
            pltpu.async_copy(onesv, acc.at[didx.at[c + j]], add=True,
                             sem=dsem)
        for j in range(8):
            pltpu.make_async_copy(onesv, acc.at[didx.at[c + j]], dsem).wait()

    plsc.subcore_barrier()
    pltpu.sync_copy(acc.at[pl.ds(sid * RPT, RPT)], zb)
    pltpu.sync_copy(zb, out_hbm.at[pl.ds(cid * NP + sid * RPT, RPT)])


@functools.partial(
    pl.kernel,
    out_type=jax.ShapeDtypeStruct((NC, NP, C), jnp.float32),
    mesh=_MESH,
    scratch_types=[
        pltpu.VMEM((2 * BLK, K), jnp.int32),
        pltpu.VMEM((2 * BLK, K), jnp.int32),
        pltpu.VMEM((K, C), jnp.float32),
        pltpu.VMEM((K, C), jnp.float32),
        pltpu.VMEM_SHARED((NP, C), jnp.float32),
        pltpu.SemaphoreType.DMA,
        pltpu.SemaphoreType.DMA,
        pltpu.SemaphoreType.DMA,
        pltpu.SemaphoreType.DMA,
    ],
)
def _agg_kernel(u_hbm, ei_hbm, out_hbm, idx0, idx1, rows0, rows1,
                acc, isem0, isem1, gsem0, gsem1):
    cid, sid, wid = _worker_id()
    zvec = jnp.zeros((16,), jnp.float32)
    rbufs = (rows0, rows1)
    gsems = (gsem0, gsem1)

    def fill(k, _):
        rows0[k // 8, pl.ds((k % 8) * 16, 16)] = zvec
        return 0

    lax.fori_loop(0, K * 8, fill, 0)

    def zcopy(j, _):
        pltpu.sync_copy(rows0, acc.at[pl.ds(sid * RPT + j * K, K)])
        return 0

    lax.fori_loop(0, RPT // K, zcopy, 0)
    plsc.subcore_barrier()

    base2 = wid * RPW * 2

    def idx_slice(b):
        return ei_hbm.at[pl.ds(base2 + b * 2 * BLK, 2 * BLK)]

    def one_block(b, ix, iy, isy):
        for j in range(BLK):
            cur = rbufs[j % 2]
            nxt = rbufs[(j + 1) % 2]
            if j < BLK - 1:
                pltpu.async_copy(u_hbm.at[ix.at[2 * (j + 1)]], nxt,
                                 gsems[(j + 1) % 2])
            else:
                @pl.when(b + 1 < NBLK)
                def _():
                    pltpu.make_async_copy(idx_slice(b + 1), iy, isy).wait()
                    pltpu.async_copy(u_hbm.at[iy.at[0]], nxt,
                                     gsems[(j + 1) % 2])
            pltpu.make_async_copy(u_hbm.at[ix.at[2 * j]], cur,
                                  gsems[j % 2]).wait()
            pltpu.sync_copy(cur, acc.at[ix.at[2 * j + 1]], add=True)

    pltpu.sync_copy(idx_slice(0), idx0)
    pltpu.async_copy(idx_slice(1), idx1, isem1)
    pltpu.async_copy(u_hbm.at[idx0.at[0]], rows0, gsem0)

    @pl.loop(0, NBLK, step=2)
    def blocks(b):
        one_block(b, idx0, idx1, isem1)

        @pl.when(b + 2 < NBLK)
        def _():
            pltpu.async_copy(idx_slice(b + 2), idx0, isem0)

        one_block(b + 1, idx1, idx0, isem0)

        @pl.when(b + 3 < NBLK)
        def _():
            pltpu.async_copy(idx_slice(b + 3), idx1, isem1)

    plsc.subcore_barrier()

    row0 = sid * RPT
    pltpu.async_copy(acc.at[pl.ds(row0, K)], rows0, gsem0)
    for j in range(RPT // K):
        cur = rbufs[j % 2]
        if j + 1 < RPT // K:
            pltpu.async_copy(acc.at[pl.ds(row0 + (j + 1) * K, K)],
                             rbufs[(j + 1) % 2], gsems[(j + 1) % 2])
        pltpu.make_async_copy(acc.at[pl.ds(row0 + j * K, K)], cur,
                              gsems[j % 2]).wait()
        pltpu.sync_copy(cur, out_hbm.at[cid, pl.ds(row0 + j * K, K)])


_BS = 1024


def _scale_body(degp_ref, x_ref, d_ref, u1_ref):
    deg = degp_ref[0] + degp_ref[1] + 1.0
    d = lax.rsqrt(deg)
    d_ref[...] = d
    u1_ref[...] = x_ref[...] * d


def _scale_call(degp, x):
    return pl.pallas_call(
        _scale_body,
        grid=(NP // _BS,),
        in_specs=[
            pl.BlockSpec((NC, _BS, 1), lambda i: (0, i, 0)),
            pl.BlockSpec((_BS, C), lambda i: (i, 0)),
        ],
        out_specs=[
            pl.BlockSpec((_BS, 1), lambda i: (i, 0)),
            pl.BlockSpec((_BS, C), lambda i: (i, 0)),
        ],
        out_shape=[
            jax.ShapeDtypeStruct((NP, 1), jnp.float32),
            jax.ShapeDtypeStruct((NP, C), jnp.float32),
        ],
    )(degp, x)


def _dense_body(s1p_ref, u1_ref, d_ref, W1_ref, b1_ref, W2_ref, u2_ref):
    t = s1p_ref[0] + s1p_ref[1] + u1_ref[...]
    d = d_ref[...]
    y = jnp.dot(t.astype(jnp.bfloat16), W1_ref[...].astype(jnp.bfloat16),
                preferred_element_type=jnp.float32) * d
    r = jnp.maximum(y + b1_ref[...], 0.0)
    u2_ref[...] = jnp.dot(r.astype(jnp.bfloat16),
                          W2_ref[...].astype(jnp.bfloat16),
                          preferred_element_type=jnp.float32) * d


def _dense_call(s1p, u1, d, W1, b1, W2):
    return pl.pallas_call(
        _dense_body,
        grid=(NP // _BS,),
        in_specs=[
            pl.BlockSpec((NC, _BS, C), lambda i: (0, i, 0)),
            pl.BlockSpec((_BS, C), lambda i: (i, 0)),
            pl.BlockSpec((_BS, 1), lambda i: (i, 0)),
            pl.BlockSpec((C, H), lambda i: (0, 0)),
            pl.BlockSpec((1, H), lambda i: (0, 0)),
            pl.BlockSpec((H, C), lambda i: (0, 0)),
        ],
        out_specs=pl.BlockSpec((_BS, C), lambda i: (i, 0)),
        out_shape=jax.ShapeDtypeStruct((NP, C), jnp.float32),
    )(s1p, u1, d, W1, b1.reshape(1, H), W2)


def _final_body(s2p_ref, u2_ref, d_ref, b2_ref, out_ref):
    out_ref[...] = ((s2p_ref[0] + s2p_ref[1] + u2_ref[...]) * d_ref[...]
                    + b2_ref[...])


def _final_call(s2p, u2, d, b2):
    return pl.pallas_call(
        _final_body,
        grid=(NP // _BS,),
        in_specs=[
            pl.BlockSpec((NC, _BS, C), lambda i: (0, i, 0)),
            pl.BlockSpec((_BS, C), lambda i: (i, 0)),
            pl.BlockSpec((_BS, 1), lambda i: (i, 0)),
            pl.BlockSpec((1, C), lambda i: (0, 0)),
        ],
        out_specs=pl.BlockSpec((_BS, C), lambda i: (i, 0)),
        out_shape=jax.ShapeDtypeStruct((NP, C), jnp.float32),
    )(s2p, u2, d, b2.reshape(1, C))


def kernel(x, edge_index, W1, b1, W2, b2):
    pad = N + jnp.arange(EP - E, dtype=jnp.int32) % (NP - N)
    src = jnp.concatenate([edge_index[0].astype(jnp.int32), pad]).reshape(ROWS, K)
    dst = jnp.concatenate([edge_index[1].astype(jnp.int32), pad]).reshape(ROWS, K)
    ei = jnp.stack([src, dst], axis=1).reshape(2 * ROWS, K)
    xp = jnp.pad(x, ((0, NP - N), (0, 0)))
    degp = _deg_kernel(dst).reshape(NC, NP, 1)
    d, u1 = _scale_call(degp, xp)
    s1p = _agg_kernel(u1, ei)
    u2 = _dense_call(s1p, u1, d, W1, b1, W2)
    s2p = _agg_kernel(u2, ei)
    return _final_call(s2p, u2, d, b2)[:N]


# speedup vs baseline: 1.1195x; 1.0026x over previous
import functools

import jax
import jax.numpy as jnp
from jax import lax
from jax.experimental import pallas as pl
from jax.experimental.pallas import tpu as pltpu
from jax.experimental.pallas import tpu_sc as plsc

N = 10000
NP = 10240
E = 320000
C = 128
H = 256

_INFO = plsc.get_sparse_core_info()
NC = _INFO.num_cores
NS = _INFO.num_subcores
NW = NC * NS
K = 128
RPW = 80
ROWS = NW * RPW
EP = ROWS * K
BLK = 4
NBLK = RPW // BLK
RPT = NP // NS

_MESH = plsc.VectorSubcoreMesh(core_axis_name="c", subcore_axis_name="s")


def _worker_id():
    cid = lax.axis_index("c")
    sid = lax.axis_index("s")
    return cid, sid, sid * NC + cid


@functools.partial(
    pl.kernel,
    out_type=jax.ShapeDtypeStruct((NC * NP,), jnp.float32),
    mesh=_MESH,
    scratch_types=[
        pltpu.VMEM((RPW, K), jnp.int32),
        pltpu.VMEM((K,), jnp.float32),
        pltpu.VMEM((RPT,), jnp.float32),
        pltpu.VMEM_SHARED((NP,), jnp.float32),
        pltpu.SemaphoreType.DMA,
    ],
)
def _deg_kernel(dst_hbm, out_hbm, didx, onesv, zb, acc, dsem):
    cid, sid, wid = _worker_id()
    zvec = jnp.zeros((16,), jnp.float32)
    ovec = jnp.ones((16,), jnp.float32)

    def fill(i, _):
        zb[pl.ds(i * 16, 16)] = zvec
        return 0

    lax.fori_loop(0, RPT // 16, fill, 0)

    def fill1(i, _):
        onesv[pl.ds(i * 16, 16)] = ovec
        return 0

    lax.fori_loop(0, K // 16, fill1, 0)
    pltpu.sync_copy(dst_hbm.at[pl.ds(wid * RPW, RPW)], didx)
    pltpu.sync_copy(zb, acc.at[pl.ds(sid * RPT, RPT)])
    plsc.subcore_barrier()

    @pl.loop(0, RPW, step=8)
    def chunk(c):
        for j in range(8):
            pltpu.async_copy(onesv, acc.at[didx.at[c + j]], add=True,
                             sem=dsem)
        for j in range(8):
            pltpu.make_async_copy(onesv, acc.at[didx.at[c + j]], dsem).wait()

    plsc.subcore_barrier()
    pltpu.sync_copy(acc.at[pl.ds(sid * RPT, RPT)], zb)
    pltpu.sync_copy(zb, out_hbm.at[pl.ds(cid * NP + sid * RPT, RPT)])


@functools.partial(
    pl.kernel,
    out_type=jax.ShapeDtypeStruct((NC, NP, C), jnp.float32),
    mesh=_MESH,
    scratch_types=[
        pltpu.VMEM((2 * BLK, K), jnp.int32),
        pltpu.VMEM((2 * BLK, K), jnp.int32),
        pltpu.VMEM((K, C), jnp.float32),
        pltpu.VMEM((K, C), jnp.float32),
        pltpu.VMEM_SHARED((NP, C), jnp.float32),
        pltpu.SemaphoreType.DMA,
        pltpu.SemaphoreType.DMA,
        pltpu.SemaphoreType.DMA,
        pltpu.SemaphoreType.DMA,
    ],
)
def _agg_kernel(u_hbm, ei_hbm, out_hbm, idx0, idx1, rows0, rows1,
                acc, isem0, isem1, gsem0, gsem1):
    cid, sid, wid = _worker_id()
    zvec = jnp.zeros((16,), jnp.float32)
    rbufs = (rows0, rows1)
    gsems = (gsem0, gsem1)

    def fill(k, _):
        rows0[k // 8, pl.ds((k % 8) * 16, 16)] = zvec
        return 0

    lax.fori_loop(0, K * 8, fill, 0)

    def zcopy(j, _):
        pltpu.sync_copy(rows0, acc.at[pl.ds(sid * RPT + j * K, K)])
        return 0

    lax.fori_loop(0, RPT // K, zcopy, 0)
    plsc.subcore_barrier()

    base2 = wid * RPW * 2

    def idx_slice(b):
        return ei_hbm.at[pl.ds(base2 + b * 2 * BLK, 2 * BLK)]

    def one_block(b, ix, iy, isy):
        for j in range(BLK):
            cur = rbufs[j % 2]
            nxt = rbufs[(j + 1) % 2]
            if j < BLK - 1:
                pltpu.async_copy(u_hbm.at[ix.at[2 * (j + 1)]], nxt,
                                 gsems[(j + 1) % 2])
            else:
                @pl.when(b + 1 < NBLK)
                def _():
                    pltpu.make_async_copy(idx_slice(b + 1), iy, isy).wait()
                    pltpu.async_copy(u_hbm.at[iy.at[0]], nxt,
                                     gsems[(j + 1) % 2])
            pltpu.make_async_copy(u_hbm.at[ix.at[2 * j]], cur,
                                  gsems[j % 2]).wait()
            pltpu.sync_copy(cur, acc.at[ix.at[2 * j + 1]], add=True)

    pltpu.sync_copy(idx_slice(0), idx0)
    pltpu.async_copy(idx_slice(1), idx1, isem1)
    pltpu.async_copy(u_hbm.at[idx0.at[0]], rows0, gsem0)

    @pl.loop(0, NBLK, step=2)
    def blocks(b):
        one_block(b, idx0, idx1, isem1)

        @pl.when(b + 2 < NBLK)
        def _():
            pltpu.async_copy(idx_slice(b + 2), idx0, isem0)

        one_block(b + 1, idx1, idx0, isem0)

        @pl.when(b + 3 < NBLK)
        def _():
            pltpu.async_copy(idx_slice(b + 3), idx1, isem1)

    plsc.subcore_barrier()

    row0 = sid * RPT
    pltpu.async_copy(acc.at[pl.ds(row0, K)], rows0, gsem0)
    for j in range(RPT // K):
        cur = rbufs[j % 2]
        if j + 1 < RPT // K:
            pltpu.async_copy(acc.at[pl.ds(row0 + (j + 1) * K, K)],
                             rbufs[(j + 1) % 2], gsems[(j + 1) % 2])
        pltpu.make_async_copy(acc.at[pl.ds(row0 + j * K, K)], cur,
                              gsems[j % 2]).wait()
        pltpu.sync_copy(cur, out_hbm.at[cid, pl.ds(row0 + j * K, K)])


_BS = 1024


def _scale_body(degp_ref, x_ref, d_ref, u1_ref):
    deg = degp_ref[0] + degp_ref[1] + 1.0
    d = lax.rsqrt(deg)
    d_ref[...] = d
    u1_ref[...] = x_ref[...] * d


def _scale_call(degp, x):
    return pl.pallas_call(
        _scale_body,
        grid=(NP // _BS,),
        in_specs=[
            pl.BlockSpec((NC, _BS, 1), lambda i: (0, i, 0)),
            pl.BlockSpec((_BS, C), lambda i: (i, 0)),
        ],
        out_specs=[
            pl.BlockSpec((_BS, 1), lambda i: (i, 0)),
            pl.BlockSpec((_BS, C), lambda i: (i, 0)),
        ],
        out_shape=[
            jax.ShapeDtypeStruct((NP, 1), jnp.float32),
            jax.ShapeDtypeStruct((NP, C), jnp.float32),
        ],
    )(degp, x)


def _dense_body(s1p_ref, u1_ref, d_ref, W1_ref, b1_ref, W2_ref, u2_ref):
    t = s1p_ref[0] + s1p_ref[1] + u1_ref[...]
    d = d_ref[...]
    y = jnp.dot(t.astype(jnp.bfloat16), W1_ref[...].astype(jnp.bfloat16),
                preferred_element_type=jnp.float32) * d
    r = jnp.maximum(y + b1_ref[...], 0.0)
    u2_ref[...] = jnp.dot(r.astype(jnp.bfloat16),
                          W2_ref[...].astype(jnp.bfloat16),
                          preferred_element_type=jnp.float32) * d


def _dense_call(s1p, u1, d, W1, b1, W2):
    return pl.pallas_call(
        _dense_body,
        grid=(NP // _BS,),
        in_specs=[
            pl.BlockSpec((NC, _BS, C), lambda i: (0, i, 0)),
            pl.BlockSpec((_BS, C), lambda i: (i, 0)),
            pl.BlockSpec((_BS, 1), lambda i: (i, 0)),
            pl.BlockSpec((C, H), lambda i: (0, 0)),
            pl.BlockSpec((1, H), lambda i: (0, 0)),
            pl.BlockSpec((H, C), lambda i: (0, 0)),
        ],
        out_specs=pl.BlockSpec((_BS, C), lambda i: (i, 0)),
        out_shape=jax.ShapeDtypeStruct((NP, C), jnp.float32),
    )(s1p, u1, d, W1, b1.reshape(1, H), W2)


def _final_body(s2p_ref, u2_ref, d_ref, b2_ref, out_ref):
    out_ref[...] = ((s2p_ref[0] + s2p_ref[1] + u2_ref[...]) * d_ref[...]
                    + b2_ref[...])


def _final_call(s2p, u2, d, b2):
    return pl.pallas_call(
        _final_body,
        grid=(NP // _BS,),
        in_specs=[
            pl.BlockSpec((NC, _BS, C), lambda i: (0, i, 0)),
            pl.BlockSpec((_BS, C), lambda i: (i, 0)),
            pl.BlockSpec((_BS, 1), lambda i: (i, 0)),
            pl.BlockSpec((1, C), lambda i: (0, 0)),
        ],
        out_specs=pl.BlockSpec((_BS, C), lambda i: (i, 0)),
        out_shape=jax.ShapeDtypeStruct((NP, C), jnp.float32),
    )(s2p, u2, d, b2.reshape(1, C))


def kernel(x, edge_index, W1, b1, W2, b2):
    pad = N + jnp.arange(EP - E, dtype=jnp.int32) % (NP - N)
    src = jnp.concatenate([edge_index[0].astype(jnp.int32), pad]).reshape(ROWS, K)
    dst = jnp.concatenate([edge_index[1].astype(jnp.int32), pad]).reshape(ROWS, K)
    ei = jnp.stack([src, dst], axis=1).reshape(2 * ROWS, K)
    xp = jnp.pad(x, ((0, NP - N), (0, 0)))
    degp = _deg_kernel(dst).reshape(NC, NP, 1)
    d, u1 = _scale_call(degp, xp)
    s1p = _agg_kernel(u1, ei)
    u2 = _dense_call(s1p, u1, d, W1, b1, W2)
    s2p = _agg_kernel(u2, ei)
    return _final_call(s2p, u2, d, b2)[:N]


# speedup vs baseline: 1.1229x; 1.0030x over previous
import functools

import jax
import jax.numpy as jnp
from jax import lax
from jax.experimental import pallas as pl
from jax.experimental.pallas import tpu as pltpu
from jax.experimental.pallas import tpu_sc as plsc

N = 10000
NP = 10240
E = 320000
C = 128
H = 256

NC = 2
NS = 16
NW = NC * NS
K = 128
RPW = 80
ROWS = NW * RPW
EP = ROWS * K
BLK = 4
NBLK = RPW // BLK
RPT = NP // NS

_MESH = plsc.VectorSubcoreMesh(core_axis_name="c", subcore_axis_name="s",
                               num_cores=NC, num_subcores=NS)


def _worker_id():
    cid = lax.axis_index("c")
    sid = lax.axis_index("s")
    return cid, sid, sid * NC + cid


@functools.partial(
    pl.kernel,
    out_type=jax.ShapeDtypeStruct((NC * NP,), jnp.float32),
    mesh=_MESH,
    scratch_types=[
        pltpu.VMEM((RPW, K), jnp.int32),
        pltpu.VMEM((K,), jnp.float32),
        pltpu.VMEM((RPT,), jnp.float32),
        pltpu.VMEM_SHARED((NP,), jnp.float32),
        pltpu.SemaphoreType.DMA,
    ],
)
def _deg_kernel(dst_hbm, out_hbm, didx, onesv, zb, acc, dsem):
    cid, sid, wid = _worker_id()
    zvec = jnp.zeros((16,), jnp.float32)
    ovec = jnp.ones((16,), jnp.float32)

    def fill(i, _):
        zb[pl.ds(i * 16, 16)] = zvec
        return 0

    lax.fori_loop(0, RPT // 16, fill, 0)

    def fill1(i, _):
        onesv[pl.ds(i * 16, 16)] = ovec
        return 0

    lax.fori_loop(0, K // 16, fill1, 0)
    pltpu.sync_copy(dst_hbm.at[pl.ds(wid * RPW, RPW)], didx)
    pltpu.sync_copy(zb, acc.at[pl.ds(sid * RPT, RPT)])
    plsc.subcore_barrier()

    @pl.loop(0, RPW, step=8)
    def chunk(c):
        for j in range(8):
            pltpu.async_copy(onesv, acc.at[didx.at[c + j]], add=True,
                             sem=dsem)
        for j in range(8):
            pltpu.make_async_copy(onesv, acc.at[didx.at[c + j]], dsem).wait()

    plsc.subcore_barrier()
    pltpu.sync_copy(acc.at[pl.ds(sid * RPT, RPT)], zb)
    pltpu.sync_copy(zb, out_hbm.at[pl.ds(cid * NP + sid * RPT, RPT)])


@functools.partial(
    pl.kernel,
    out_type=jax.ShapeDtypeStruct((NC, NP, C), jnp.float32),
    mesh=_MESH,
    scratch_types=[
        pltpu.VMEM((2 * BLK, K), jnp.int32),
        pltpu.VMEM((2 * BLK, K), jnp.int32),
        pltpu.VMEM((K, C), jnp.float32),
        pltpu.VMEM((K, C), jnp.float32),
        pltpu.VMEM_SHARED((NP, C), jnp.float32),
        pltpu.SemaphoreType.DMA,
        pltpu.SemaphoreType.DMA,
        pltpu.SemaphoreType.DMA,
        pltpu.SemaphoreType.DMA,
    ],
)
def _agg_kernel(u_hbm, ei_hbm, out_hbm, idx0, idx1, rows0, rows1,
                acc, isem0, isem1, gsem0, gsem1):
    cid, sid, wid = _worker_id()
    zvec = jnp.zeros((16,), jnp.float32)
    rbufs = (rows0, rows1)
    gsems = (gsem0, gsem1)

    def fill(k, _):
        rows0[k // 8, pl.ds((k % 8) * 16, 16)] = zvec
        return 0

    lax.fori_loop(0, K * 8, fill, 0)

    def zcopy(j, _):
        pltpu.sync_copy(rows0, acc.at[pl.ds(sid * RPT + j * K, K)])
        return 0

    lax.fori_loop(0, RPT // K, zcopy, 0)
    plsc.subcore_barrier()

    base2 = wid * RPW * 2

    def idx_slice(b):
        return ei_hbm.at[pl.ds(base2 + b * 2 * BLK, 2 * BLK)]

    def one_block(b, ix, iy, isy):
        for j in range(BLK):
            cur = rbufs[j % 2]
            nxt = rbufs[(j + 1) % 2]
            if j < BLK - 1:
                pltpu.async_copy(u_hbm.at[ix.at[2 * (j + 1)]], nxt,
                                 gsems[(j + 1) % 2])
            else:
                @pl.when(b + 1 < NBLK)
                def _():
                    pltpu.make_async_copy(idx_slice(b + 1), iy, isy).wait()
                    pltpu.async_copy(u_hbm.at[iy.at[0]], nxt,
                                     gsems[(j + 1) % 2])
            pltpu.make_async_copy(u_hbm.at[ix.at[2 * j]], cur,
                                  gsems[j % 2]).wait()
            pltpu.sync_copy(cur, acc.at[ix.at[2 * j + 1]], add=True)

    pltpu.sync_copy(idx_slice(0), idx0)
    pltpu.async_copy(idx_slice(1), idx1, isem1)
    pltpu.async_copy(u_hbm.at[idx0.at[0]], rows0, gsem0)

    @pl.loop(0, NBLK, step=2)
    def blocks(b):
        one_block(b, idx0, idx1, isem1)

        @pl.when(b + 2 < NBLK)
        def _():
            pltpu.async_copy(idx_slice(b + 2), idx0, isem0)

        one_block(b + 1, idx1, idx0, isem0)

        @pl.when(b + 3 < NBLK)
        def _():
            pltpu.async_copy(idx_slice(b + 3), idx1, isem1)

    plsc.subcore_barrier()

    row0 = sid * RPT
    pltpu.async_copy(acc.at[pl.ds(row0, K)], rows0, gsem0)
    for j in range(RPT // K):
        cur = rbufs[j % 2]
        if j + 1 < RPT // K:
            pltpu.async_copy(acc.at[pl.ds(row0 + (j + 1) * K, K)],
                             rbufs[(j + 1) % 2], gsems[(j + 1) % 2])
        pltpu.make_async_copy(acc.at[pl.ds(row0 + j * K, K)], cur,
                              gsems[j % 2]).wait()
        pltpu.sync_copy(cur, out_hbm.at[cid, pl.ds(row0 + j * K, K)])


_BS = 1024


def _scale_body(degp_ref, x_ref, d_ref, u1_ref):
    deg = degp_ref[0] + degp_ref[1] + 1.0
    d = lax.rsqrt(deg)
    d_ref[...] = d
    u1_ref[...] = x_ref[...] * d


def _scale_call(degp, x):
    return pl.pallas_call(
        _scale_body,
        grid=(NP // _BS,),
        in_specs=[
            pl.BlockSpec((NC, _BS, 1), lambda i: (0, i, 0)),
            pl.BlockSpec((_BS, C), lambda i: (i, 0)),
        ],
        out_specs=[
            pl.BlockSpec((_BS, 1), lambda i: (i, 0)),
            pl.BlockSpec((_BS, C), lambda i: (i, 0)),
        ],
        out_shape=[
            jax.ShapeDtypeStruct((NP, 1), jnp.float32),
            jax.ShapeDtypeStruct((NP, C), jnp.float32),
        ],
    )(degp, x)


def _dense_body(s1p_ref, u1_ref, d_ref, W1_ref, b1_ref, W2_ref, u2_ref):
    t = s1p_ref[0] + s1p_ref[1] + u1_ref[...]
    d = d_ref[...]
    y = jnp.dot(t.astype(jnp.bfloat16), W1_ref[...].astype(jnp.bfloat16),
                preferred_element_type=jnp.float32) * d
    r = jnp.maximum(y + b1_ref[...], 0.0)
    u2_ref[...] = jnp.dot(r.astype(jnp.bfloat16),
                          W2_ref[...].astype(jnp.bfloat16),
                          preferred_element_type=jnp.float32) * d


def _dense_call(s1p, u1, d, W1, b1, W2):
    return pl.pallas_call(
        _dense_body,
        grid=(NP // _BS,),
        in_specs=[
            pl.BlockSpec((NC, _BS, C), lambda i: (0, i, 0)),
            pl.BlockSpec((_BS, C), lambda i: (i, 0)),
            pl.BlockSpec((_BS, 1), lambda i: (i, 0)),
            pl.BlockSpec((C, H), lambda i: (0, 0)),
            pl.BlockSpec((1, H), lambda i: (0, 0)),
            pl.BlockSpec((H, C), lambda i: (0, 0)),
        ],
        out_specs=pl.BlockSpec((_BS, C), lambda i: (i, 0)),
        out_shape=jax.ShapeDtypeStruct((NP, C), jnp.float32),
    )(s1p, u1, d, W1, b1.reshape(1, H), W2)


def _final_body(s2p_ref, u2_ref, d_ref, b2_ref, out_ref):
    out_ref[...] = ((s2p_ref[0] + s2p_ref[1] + u2_ref[...]) * d_ref[...]
                    + b2_ref[...])


def _final_call(s2p, u2, d, b2):
    return pl.pallas_call(
        _final_body,
        grid=(NP // _BS,),
        in_specs=[
            pl.BlockSpec((NC, _BS, C), lambda i: (0, i, 0)),
            pl.BlockSpec((_BS, C), lambda i: (i, 0)),
            pl.BlockSpec((_BS, 1), lambda i: (i, 0)),
            pl.BlockSpec((1, C), lambda i: (0, 0)),
        ],
        out_specs=pl.BlockSpec((_BS, C), lambda i: (i, 0)),
        out_shape=jax.ShapeDtypeStruct((NP, C), jnp.float32),
    )(s2p, u2, d, b2.reshape(1, C))


def kernel(x, edge_index, W1, b1, W2, b2):
    pad = N + jnp.arange(EP - E, dtype=jnp.int32) % (NP - N)
    src = jnp.concatenate([edge_index[0].astype(jnp.int32), pad]).reshape(ROWS, K)
    dst = jnp.concatenate([edge_index[1].astype(jnp.int32), pad]).reshape(ROWS, K)
    ei = jnp.stack([src, dst], axis=1).reshape(2 * ROWS, K)
    xp = jnp.pad(x, ((0, NP - N), (0, 0)))
    degp = _deg_kernel(dst).reshape(NC, NP, 1)
    d, u1 = _scale_call(degp, xp)
    s1p = _agg_kernel(u1, ei)
    u2 = _dense_call(s1p, u1, d, W1, b1, W2)
    s2p = _agg_kernel(u2, ei)
    return _final_call(s2p, u2, d, b2)[:N]
